# baseline (speedup 1.0000x reference)
import functools

import jax
import jax.numpy as jnp
from jax import lax
from jax.experimental import pallas as pl
from jax.experimental.pallas import tpu as pltpu
from jax.experimental.pallas import tpu_sc as plsc


def _sc_gather(emb_table, xf, sf):
    n_tok = xf.shape[0]
    v, d = emb_table.shape
    info = plsc.get_sparse_core_info()
    nc, ns = info.num_cores, info.num_subcores
    nw = nc * ns
    assert n_tok % nw == 0
    b_per_w = n_tok // nw

    mesh = plsc.VectorSubcoreMesh(core_axis_name="c", subcore_axis_name="s")

    @functools.partial(
        pl.kernel,
        mesh=mesh,
        out_type=[
            jax.ShapeDtypeStruct((n_tok, d), jnp.float32),
            jax.ShapeDtypeStruct((n_tok, d), jnp.float32),
        ],
        scratch_types=[
            pltpu.VMEM((b_per_w,), jnp.int32),
            pltpu.VMEM((b_per_w,), jnp.int32),
            pltpu.VMEM((b_per_w, d), jnp.float32),
            pltpu.VMEM((b_per_w, d), jnp.float32),
            pltpu.SemaphoreType.DMA,
        ],
        compiler_params=pltpu.CompilerParams(use_tc_tiling_on_sc=False),
    )
    def body2(emb_hbm, xf_hbm, sf_hbm, e1_hbm, e2_hbm, xi_v, si_v, r1_v, r2_v, sem):
        wid = lax.axis_index("s") * nc + lax.axis_index("c")
        base = wid * b_per_w
        pltpu.sync_copy(xf_hbm.at[pl.ds(base, b_per_w)], xi_v)
        pltpu.sync_copy(sf_hbm.at[pl.ds(base, b_per_w)], si_v)
        c1 = pltpu.async_copy(emb_hbm.at[xi_v], r1_v, sem)
        c2 = pltpu.async_copy(emb_hbm.at[si_v], r2_v, sem)
        c1.wait()
        c2.wait()
        pltpu.sync_copy(r1_v, e1_hbm.at[pl.ds(base, b_per_w)])
        pltpu.sync_copy(r2_v, e2_hbm.at[pl.ds(base, b_per_w)])

    return body2(emb_table, xf, sf)


def _tc_project(e1, e2, w1, w2, b2, m_blk=512):
    n_tok, d = e1.shape
    vocab = w1.shape[1]
    assert n_tok % m_blk == 0

    def body(e1_ref, e2_ref, w1_ref, w2_ref, b_ref, out_ref):
        acc = jnp.dot(e1_ref[...], w1_ref[...], preferred_element_type=jnp.float32)
        acc = acc + jnp.dot(e2_ref[...], w2_ref[...], preferred_element_type=jnp.float32)
        out_ref[...] = acc + b_ref[...]

    return pl.pallas_call(
        body,
        grid=(n_tok // m_blk,),
        in_specs=[
            pl.BlockSpec((m_blk, d), lambda i: (i, 0)),
            pl.BlockSpec((m_blk, d), lambda i: (i, 0)),
            pl.BlockSpec((d, vocab), lambda i: (0, 0)),
            pl.BlockSpec((d, vocab), lambda i: (0, 0)),
            pl.BlockSpec((1, vocab), lambda i: (0, 0)),
        ],
        out_specs=pl.BlockSpec((m_blk, vocab), lambda i: (i, 0)),
        out_shape=jax.ShapeDtypeStruct((n_tok, vocab), jnp.float32),
        compiler_params=pltpu.CompilerParams(
            dimension_semantics=("parallel",),
        ),
    )(e1, e2, w1, w2, b2)


def kernel(x, emb_table, W, b):
    bsz, t = x.shape
    v, d = emb_table.shape
    x = x.astype(jnp.int32)
    sx = jnp.concatenate(
        (jnp.zeros((bsz, 1), dtype=x.dtype), x[:, :-1]), axis=1
    )
    xf = x.reshape(-1)
    sf = sx.reshape(-1)
    e1, e2 = _sc_gather(emb_table, xf, sf)
    logits = _tc_project(e1, e2, W[:d], W[d:], b.reshape(1, -1))
    return logits.reshape(bsz, t, v)
